# baseline (speedup 1.0000x reference)
import jax
import jax.numpy as jnp
from jax import lax
from jax.experimental import pallas as pl
from jax.experimental.pallas import tpu as pltpu
from jax.experimental.pallas import tpu_sc as plsc

B = 4096
F = 26
V = 100000
D = 64
ND = 13

VT = 32768
NVT = 4
TAIL = V - (NVT - 1) * VT
TA = 1664
TB = TAIL - TA
HB = VT // 2
HT = TAIL // 2
RPF = V // 2
NP = F * RPF

NC = 2
NS = 16
NW = NC * NS
RPW = B // NW
CH = 16
NCH = RPW // CH
M = 32
GPC = CH * F // M
NG = CH * F // 16
ND4 = D // 16


def _tc_body(dense_ref, w2_ref, b2_ref, out_ref):
    d = dense_ref[...]
    dn = (((1,), (1,)), ((), ()))
    out_ref[...] = (
        lax.dot_general(d, w2_ref[...], dn, preferred_element_type=jnp.float32)
        + b2_ref[...][None, :])


def _dense_stage(dense_features, W_arch, b_arch, W_lin, b_lin, bias):
    w2 = jnp.concatenate([W_arch, W_lin], axis=0)
    b2 = jnp.concatenate([b_arch, b_lin + bias[0]], axis=0)
    out = pl.pallas_call(
        _tc_body,
        out_shape=jax.ShapeDtypeStruct((B, D + 1), jnp.float32),
    )(dense_features, w2, b2)
    return out[:, :D], out[:, D]


def _tp_body(emb_any, tail_any, out_any, inb, inb_a, inb_b, outb, outb_t,
             isem, osem, tsem):
    f = pl.program_id(0)
    vt = pl.program_id(1)
    i = f * NVT + vt
    row0 = f * RPF + vt * HB

    def start_in(j, slot):
        fj = j // NVT
        vj = j - fj * NVT
        pltpu.make_async_copy(
            emb_any.at[fj, :, pl.ds(vj * VT, VT)], inb.at[slot],
            isem.at[slot]).start()

    is_full = vt < NVT - 1

    @pl.when(i == 0)
    def _first():
        start_in(0, 0)

    nxt = i + 1
    fn_ = nxt // NVT
    vn = nxt - fn_ * NVT

    @pl.when((nxt < F * NVT) & (vn < NVT - 1))
    def _prefetch():
        start_in(nxt, nxt % 2)

    @pl.when(is_full)
    def _full():
        slot = i % 2
        pltpu.make_async_copy(
            emb_any.at[f, :, pl.ds(vt * VT, VT)], inb.at[slot],
            isem.at[slot]).wait()

        @pl.when(i >= 2)
        def _drain():
            pltpu.make_async_copy(
                outb.at[slot], out_any.at[pl.ds(row0, HB)],
                osem.at[slot]).wait()

        xT = jnp.transpose(inb[slot])
        outb[slot] = jnp.concatenate([xT[:HB], xT[HB:]], axis=1)
        pltpu.make_async_copy(
            outb.at[slot], out_any.at[pl.ds(row0, HB)],
            osem.at[slot]).start()

    @pl.when(jnp.logical_not(is_full))
    def _tail():
        ha = pltpu.make_async_copy(
            emb_any.at[f, :, pl.ds(vt * VT, TA)], inb_a, isem.at[0])
        hb = pltpu.make_async_copy(tail_any.at[f], inb_b, tsem)
        ha.start()
        hb.start()
        ha.wait()
        hb.wait()
        xTa = jnp.transpose(inb_a[...])
        xTb = jnp.transpose(inb_b[...])
        left = xTa[:HT]
        right = jnp.concatenate([xTa[HT:], xTb], axis=0)
        outb_t[...] = jnp.concatenate([left, right], axis=1)
        ht = pltpu.make_async_copy(
            outb_t, out_any.at[pl.ds(row0, HT)], tsem)
        ht.start()
        ht.wait()

        @pl.when(i == F * NVT - 1)
        def _final_drain():
            pltpu.make_async_copy(
                outb.at[0], out_any.at[pl.ds(row0, HB)], osem.at[0]).wait()
            pltpu.make_async_copy(
                outb.at[1], out_any.at[pl.ds(row0, HB)], osem.at[1]).wait()


def _tc_transpose(emb):
    emb_t = jnp.transpose(emb, (0, 2, 1))
    emb_tail = lax.slice(emb_t, (0, 0, V - TB), (F, D, V))
    return pl.pallas_call(
        _tp_body,
        grid=(F, NVT),
        in_specs=[pl.BlockSpec(memory_space=pl.ANY),
                  pl.BlockSpec(memory_space=pl.ANY)],
        out_specs=pl.BlockSpec(memory_space=pl.ANY),
        out_shape=jax.ShapeDtypeStruct((NP, 128), jnp.float32),
        scratch_shapes=[
            pltpu.VMEM((2, D, VT), jnp.float32),
            pltpu.VMEM((D, TA), jnp.float32),
            pltpu.VMEM((D, TB), jnp.float32),
            pltpu.VMEM((2, HB, 128), jnp.float32),
            pltpu.VMEM((HT, 128), jnp.float32),
            pltpu.SemaphoreType.DMA((2,)),
            pltpu.SemaphoreType.DMA((2,)),
            pltpu.SemaphoreType.DMA,
        ],
    )(emb_t, emb_tail)


def _sc_body(sf_hbm, embp_hbm, lin_hbm, darch_hbm, lind_hbm, out_hbm,
             idx_a, idx_b, lidx_a, lidx_b, rows_a, rows_b,
             linr_a, linr_b, da_a, da_b, ld_a, ld_b, sfv_a, sfv_b,
             outv, totv, sem_a, sem_b):
    wid = lax.axis_index("s") * NC + lax.axis_index("c")
    base_row = wid * RPW
    i16 = lax.iota(jnp.int32, 16)

    bufs = ((idx_a, lidx_a, rows_a, linr_a, da_a, ld_a, sfv_a, sem_a),
            (idx_b, lidx_b, rows_b, linr_b, da_b, ld_b, sfv_b, sem_b))

    def prep(c):
        idx_v, lidx_v, rows_v, linr_v, da_v, ld_v, sfv, sem = \
            bufs[c % 2]
        row0 = base_row + c * CH
        pltpu.sync_copy(sf_hbm.at[pl.ds(row0 * F, CH * F)], sfv)
        for j in range(NG):
            p = i16 + (16 * j)
            f = p - (p // F) * F
            v = sfv[pl.ds(16 * j, 16)]
            g = j // 2
            lane = (j % 2) * 16
            lidx_v[g, pl.ds(lane, 16)] = v + f * V
            vt = v >> 15
            vloc = v & 32767
            row_m = (vt << 14) + (vloc & 16383)
            half_m = vloc >> 14
            vloc_t = v - 98304
            half_t = jnp.where(vloc_t >= HT, 1, 0)
            row_t = 49152 + vloc_t - half_t * HT
            is_tail = v >= 98304
            row = f * RPF + jnp.where(is_tail, row_t, row_m)
            half = jnp.where(is_tail, half_t, half_m)
            idx_v[g, pl.ds(lane, 16)] = 2 * row + half
        hs = []
        for g in range(GPC):
            hs.append(pltpu.async_copy(
                embp_hbm.at[idx_v.at[g]], rows_v.at[pl.ds(g * M, M)], sem))
            hs.append(pltpu.async_copy(
                lin_hbm.at[lidx_v.at[g]], linr_v.at[pl.ds(g * M, M)], sem))
        hs.append(pltpu.async_copy(darch_hbm.at[pl.ds(row0, CH)], da_v, sem))
        hs.append(pltpu.async_copy(lind_hbm.at[pl.ds(row0, CH)], ld_v, sem))
        return hs

    def compute(c):
        idx_v, lidx_v, rows_v, linr_v, da_v, ld_v, sfv, sem = \
            bufs[c % 2]
        row0 = base_row + c * CH

        def rbody(r, carry):
            s = [da_v[r, pl.ds(16 * d, 16)] for d in range(ND4)]
            q = [x * x for x in s]
            for f in range(F):
                for d in range(ND4):
                    v = rows_v[r * F + f, pl.ds(16 * d, 16)]
                    s[d] = s[d] + v
                    q[d] = q[d] + v * v
            tot = (s[0] * s[0] + s[1] * s[1] + s[2] * s[2] + s[3] * s[3]
                   - (q[0] + q[1] + q[2] + q[3]))
            totv[pl.ds(r * 16, 16)] = 0.5 * tot
            return carry

        lax.fori_loop(0, CH, rbody, 0, unroll=False)

        i26 = i16 * F
        i256 = i16 * 16
        acc = ld_v[...]
        for f in range(F):
            acc = acc + plsc.load_gather(linr_v, [i26 + f])
        for k in range(16):
            acc = acc + plsc.load_gather(totv, [i256 + k])
        outv[...] = acc
        pltpu.sync_copy(outv, out_hbm.at[pl.ds(row0, CH)])

    pending = prep(0)
    for c in range(NCH):
        nxt = prep(c + 1) if c + 1 < NCH else []
        for h in pending:
            h.wait()
        pending = nxt
        compute(c)


_sc_call = pl.kernel(
    _sc_body,
    mesh=plsc.VectorSubcoreMesh(core_axis_name="c", subcore_axis_name="s"),
    compiler_params=pltpu.CompilerParams(
        use_tc_tiling_on_sc=False, needs_layout_passes=False),
    out_type=jax.ShapeDtypeStruct((B,), jnp.float32),
    scratch_types=[
        pltpu.VMEM((GPC, M), jnp.int32),
        pltpu.VMEM((GPC, M), jnp.int32),
        pltpu.VMEM((GPC, M), jnp.int32),
        pltpu.VMEM((GPC, M), jnp.int32),
        pltpu.VMEM((CH * F, D), jnp.float32),
        pltpu.VMEM((CH * F, D), jnp.float32),
        pltpu.VMEM((CH * F,), jnp.float32),
        pltpu.VMEM((CH * F,), jnp.float32),
        pltpu.VMEM((CH, D), jnp.float32),
        pltpu.VMEM((CH, D), jnp.float32),
        pltpu.VMEM((CH,), jnp.float32),
        pltpu.VMEM((CH,), jnp.float32),
        pltpu.VMEM((CH * F,), jnp.int32),
        pltpu.VMEM((CH * F,), jnp.int32),
        pltpu.VMEM((CH,), jnp.float32),
        pltpu.VMEM((CH * 16,), jnp.float32),
        pltpu.SemaphoreType.DMA,
        pltpu.SemaphoreType.DMA,
    ],
)


def kernel(sparse_features, dense_features, linear_tables, sparse_tables,
           W_lin, b_lin, W_arch, b_arch, bias):
    sf_flat = sparse_features.reshape(B * F)
    lin1 = linear_tables.reshape(F * V)
    embp = _tc_transpose(sparse_tables).reshape(2 * NP, D)
    darch, lind = _dense_stage(dense_features, W_arch, b_arch,
                               W_lin, b_lin, bias)
    out = _sc_call(sf_flat, embp, lin1, darch, lind)
    return out.reshape(B, 1)


# speedup vs baseline: 1.0350x; 1.0350x over previous
import jax
import jax.numpy as jnp
from jax import lax
from jax.experimental import pallas as pl
from jax.experimental.pallas import tpu as pltpu
from jax.experimental.pallas import tpu_sc as plsc

B = 4096
F = 26
V = 100000
D = 64
ND = 13

VT = 16384
NVT = 7
TAIL = V - (NVT - 1) * VT
TA = 1664
TB = TAIL - TA
HB = VT // 2
HT = TAIL // 2
RPF = V // 2
NP = F * RPF

NC = 2
NS = 16
NW = NC * NS
RPW = B // NW
CH = 16
NCH = RPW // CH
M = 32
GPC = CH * F // M
NG = CH * F // 16
ND4 = D // 16


def _tc_body(dense_ref, w2_ref, b2_ref, out_ref):
    d = dense_ref[...]
    dn = (((1,), (1,)), ((), ()))
    out_ref[...] = (
        lax.dot_general(d, w2_ref[...], dn, preferred_element_type=jnp.float32)
        + b2_ref[...][None, :])


def _dense_stage(dense_features, W_arch, b_arch, W_lin, b_lin, bias):
    w2 = jnp.concatenate([W_arch, W_lin], axis=0)
    b2 = jnp.concatenate([b_arch, b_lin + bias[0]], axis=0)
    out = pl.pallas_call(
        _tc_body,
        out_shape=jax.ShapeDtypeStruct((B, D + 1), jnp.float32),
    )(dense_features, w2, b2)
    return out[:, :D], out[:, D]


def _tp_body(emb_any, tail_any, out_any, inb, inb_a, inb_b, outb, outb_t,
             isem, osem, tsem):
    f = pl.program_id(0)
    vt = pl.program_id(1)
    i = f * NVT + vt
    row0 = f * RPF + vt * HB

    def start_in(j, slot):
        fj = j // NVT
        vj = j - fj * NVT
        pltpu.make_async_copy(
            emb_any.at[fj, :, pl.ds(vj * VT, VT)], inb.at[slot],
            isem.at[slot]).start()

    is_full = vt < NVT - 1

    @pl.when(i == 0)
    def _first():
        start_in(0, 0)

    nxt = i + 1
    fn_ = nxt // NVT
    vn = nxt - fn_ * NVT

    @pl.when((nxt < F * NVT) & (vn < NVT - 1))
    def _prefetch():
        start_in(nxt, nxt % 2)

    @pl.when(is_full)
    def _full():
        slot = i % 2
        pltpu.make_async_copy(
            emb_any.at[f, :, pl.ds(vt * VT, VT)], inb.at[slot],
            isem.at[slot]).wait()

        @pl.when(i >= 2)
        def _drain():
            pltpu.make_async_copy(
                outb.at[slot], out_any.at[pl.ds(row0, HB)],
                osem.at[slot]).wait()

        xT = jnp.transpose(inb[slot])
        outb[slot] = jnp.concatenate([xT[:HB], xT[HB:]], axis=1)
        pltpu.make_async_copy(
            outb.at[slot], out_any.at[pl.ds(row0, HB)],
            osem.at[slot]).start()

    @pl.when(jnp.logical_not(is_full))
    def _tail():
        ha = pltpu.make_async_copy(
            emb_any.at[f, :, pl.ds(vt * VT, TA)], inb_a, isem.at[0])
        hb = pltpu.make_async_copy(tail_any.at[f], inb_b, tsem)
        ha.start()
        hb.start()
        ha.wait()
        hb.wait()
        xTa = jnp.transpose(inb_a[...])
        xTb = jnp.transpose(inb_b[...])
        left = xTa[:HT]
        right = jnp.concatenate([xTa[HT:], xTb], axis=0)
        outb_t[...] = jnp.concatenate([left, right], axis=1)
        ht = pltpu.make_async_copy(
            outb_t, out_any.at[pl.ds(row0, HT)], tsem)
        ht.start()
        ht.wait()

        @pl.when(i == F * NVT - 1)
        def _final_drain():
            pltpu.make_async_copy(
                outb.at[0], out_any.at[pl.ds(row0, HB)], osem.at[0]).wait()
            pltpu.make_async_copy(
                outb.at[1], out_any.at[pl.ds(row0, HB)], osem.at[1]).wait()


def _tc_transpose(emb):
    emb_t = jnp.transpose(emb, (0, 2, 1))
    emb_tail = lax.slice(emb_t, (0, 0, V - TB), (F, D, V))
    return pl.pallas_call(
        _tp_body,
        grid=(F, NVT),
        in_specs=[pl.BlockSpec(memory_space=pl.ANY),
                  pl.BlockSpec(memory_space=pl.ANY)],
        out_specs=pl.BlockSpec(memory_space=pl.ANY),
        out_shape=jax.ShapeDtypeStruct((NP, 128), jnp.float32),
        scratch_shapes=[
            pltpu.VMEM((2, D, VT), jnp.float32),
            pltpu.VMEM((D, TA), jnp.float32),
            pltpu.VMEM((D, TB), jnp.float32),
            pltpu.VMEM((2, HB, 128), jnp.float32),
            pltpu.VMEM((HT, 128), jnp.float32),
            pltpu.SemaphoreType.DMA((2,)),
            pltpu.SemaphoreType.DMA((2,)),
            pltpu.SemaphoreType.DMA,
        ],
    )(emb_t, emb_tail)


def _sc_body(sf_hbm, embp_hbm, lin_hbm, darch_hbm, lind_hbm, out_hbm,
             idx_a, idx_b, lidx_a, lidx_b, rows_a, rows_b,
             linr_a, linr_b, da_a, da_b, ld_a, ld_b, sfv_a, sfv_b,
             outv, totv, sem_a, sem_b):
    wid = lax.axis_index("s") * NC + lax.axis_index("c")
    base_row = wid * RPW
    i16 = lax.iota(jnp.int32, 16)

    bufs = ((idx_a, lidx_a, rows_a, linr_a, da_a, ld_a, sfv_a, sem_a),
            (idx_b, lidx_b, rows_b, linr_b, da_b, ld_b, sfv_b, sem_b))

    def prep(c):
        idx_v, lidx_v, rows_v, linr_v, da_v, ld_v, sfv, sem = \
            bufs[c % 2]
        row0 = base_row + c * CH
        pltpu.sync_copy(sf_hbm.at[pl.ds(row0 * F, CH * F)], sfv)
        for j in range(NG):
            p = i16 + (16 * j)
            f = p - (p // F) * F
            v = sfv[pl.ds(16 * j, 16)]
            g = j // 2
            lane = (j % 2) * 16
            lidx_v[g, pl.ds(lane, 16)] = v + f * V
            vt = v >> 14
            vloc = v & 16383
            row_m = (vt << 13) + (vloc & 8191)
            half_m = vloc >> 13
            vloc_t = v - 98304
            half_t = jnp.where(vloc_t >= HT, 1, 0)
            row_t = 49152 + vloc_t - half_t * HT
            is_tail = v >= 98304
            row = f * RPF + jnp.where(is_tail, row_t, row_m)
            half = jnp.where(is_tail, half_t, half_m)
            idx_v[g, pl.ds(lane, 16)] = 2 * row + half
        hs = []
        for g in range(GPC):
            hs.append(pltpu.async_copy(
                embp_hbm.at[idx_v.at[g]], rows_v.at[pl.ds(g * M, M)], sem))
            hs.append(pltpu.async_copy(
                lin_hbm.at[lidx_v.at[g]], linr_v.at[pl.ds(g * M, M)], sem))
        hs.append(pltpu.async_copy(darch_hbm.at[pl.ds(row0, CH)], da_v, sem))
        hs.append(pltpu.async_copy(lind_hbm.at[pl.ds(row0, CH)], ld_v, sem))
        return hs

    def compute(c):
        idx_v, lidx_v, rows_v, linr_v, da_v, ld_v, sfv, sem = \
            bufs[c % 2]
        row0 = base_row + c * CH

        def rbody(r, carry):
            s = [da_v[r, pl.ds(16 * d, 16)] for d in range(ND4)]
            q = [x * x for x in s]
            for f in range(F):
                for d in range(ND4):
                    v = rows_v[r * F + f, pl.ds(16 * d, 16)]
                    s[d] = s[d] + v
                    q[d] = q[d] + v * v
            tot = (s[0] * s[0] + s[1] * s[1] + s[2] * s[2] + s[3] * s[3]
                   - (q[0] + q[1] + q[2] + q[3]))
            totv[pl.ds(r * 16, 16)] = 0.5 * tot
            return carry

        lax.fori_loop(0, CH, rbody, 0, unroll=False)

        i26 = i16 * F
        i256 = i16 * 16
        acc = ld_v[...]
        for f in range(F):
            acc = acc + plsc.load_gather(linr_v, [i26 + f])
        for k in range(16):
            acc = acc + plsc.load_gather(totv, [i256 + k])
        outv[...] = acc
        pltpu.sync_copy(outv, out_hbm.at[pl.ds(row0, CH)])

    pending = prep(0)
    for c in range(NCH):
        nxt = prep(c + 1) if c + 1 < NCH else []
        for h in pending:
            h.wait()
        pending = nxt
        compute(c)


_sc_call = pl.kernel(
    _sc_body,
    mesh=plsc.VectorSubcoreMesh(core_axis_name="c", subcore_axis_name="s"),
    compiler_params=pltpu.CompilerParams(
        use_tc_tiling_on_sc=False, needs_layout_passes=False),
    out_type=jax.ShapeDtypeStruct((B,), jnp.float32),
    scratch_types=[
        pltpu.VMEM((GPC, M), jnp.int32),
        pltpu.VMEM((GPC, M), jnp.int32),
        pltpu.VMEM((GPC, M), jnp.int32),
        pltpu.VMEM((GPC, M), jnp.int32),
        pltpu.VMEM((CH * F, D), jnp.float32),
        pltpu.VMEM((CH * F, D), jnp.float32),
        pltpu.VMEM((CH * F,), jnp.float32),
        pltpu.VMEM((CH * F,), jnp.float32),
        pltpu.VMEM((CH, D), jnp.float32),
        pltpu.VMEM((CH, D), jnp.float32),
        pltpu.VMEM((CH,), jnp.float32),
        pltpu.VMEM((CH,), jnp.float32),
        pltpu.VMEM((CH * F,), jnp.int32),
        pltpu.VMEM((CH * F,), jnp.int32),
        pltpu.VMEM((CH,), jnp.float32),
        pltpu.VMEM((CH * 16,), jnp.float32),
        pltpu.SemaphoreType.DMA,
        pltpu.SemaphoreType.DMA,
    ],
)


def kernel(sparse_features, dense_features, linear_tables, sparse_tables,
           W_lin, b_lin, W_arch, b_arch, bias):
    sf_flat = sparse_features.reshape(B * F)
    lin1 = linear_tables.reshape(F * V)
    embp = _tc_transpose(sparse_tables).reshape(2 * NP, D)
    darch, lind = _dense_stage(dense_features, W_arch, b_arch,
                               W_lin, b_lin, bias)
    out = _sc_call(sf_flat, embp, lin1, darch, lind)
    return out.reshape(B, 1)
